# baseline (speedup 1.0000x reference)
import functools

import jax
import jax.numpy as jnp
from jax import lax
from jax.experimental import pallas as pl
from jax.experimental.pallas import tpu as pltpu
from jax.experimental.pallas import tpu_sc as plsc


def _sc_gather(table, idx2d):
    rows_total = idx2d.shape[0] // 128
    d = table.shape[1]
    n_workers = 32
    rows_w = rows_total // n_workers
    streams = 5
    chunk_rows = streams * 128
    n_chunks = rows_w // streams
    n = rows_total * 128

    mesh = plsc.VectorSubcoreMesh(core_axis_name="c", subcore_axis_name="s")

    @functools.partial(
        pl.kernel,
        mesh=mesh,
        out_type=jax.ShapeDtypeStruct((n, d), table.dtype),
        scratch_types=[
            pltpu.VMEM((rows_w * 128,), jnp.int32),
            pltpu.VMEM((2 * chunk_rows, d), table.dtype),
            pltpu.SemaphoreType.DMA,
            pltpu.SemaphoreType.DMA,
        ],
        compiler_params=pltpu.CompilerParams(use_tc_tiling_on_sc=False),
    )
    def gather_kernel(idx_hbm, tab_hbm, out_hbm, idx_v, rows_v, sem_g, sem_s):
        wid = lax.axis_index("s") * 2 + lax.axis_index("c")
        out_base = wid * rows_w * 128
        pltpu.sync_copy(idx_hbm.at[pl.ds(out_base, rows_w * 128)], idx_v)

        def scatter_desc(i, cur):
            return pltpu.make_async_copy(
                rows_v.at[pl.ds(cur * chunk_rows, chunk_rows)],
                out_hbm.at[pl.ds(out_base + i * chunk_rows, chunk_rows)],
                sem_s)

        def body(i, carry):
            cur = lax.rem(i, 2)

            @pl.when(i >= 2)
            def _():
                scatter_desc(i - 2, cur).wait()

            handles = [
                pltpu.async_copy(
                    tab_hbm.at[idx_v.at[pl.ds((i * streams + j) * 128, 128)]],
                    rows_v.at[pl.ds(cur * chunk_rows + j * 128, 128)],
                    sem_g)
                for j in range(streams)
            ]
            for hnd in handles:
                hnd.wait()
            scatter_desc(i, cur).start()
            return carry

        lax.fori_loop(0, n_chunks, body, 0)
        scatter_desc(n_chunks - 2, (n_chunks - 2) % 2).wait()
        scatter_desc(n_chunks - 1, (n_chunks - 1) % 2).wait()

    return gather_kernel(idx2d, table)


def _sigmoid(x):
    return 0.5 * jnp.tanh(0.5 * x) + 0.5


_UT = 8


def _bilstm_body(xf_ref, xb_ref, wcf, wcb, wlfr, wlbr, blr,
                 outf, outb, xhf, xhb, cf, cb):
    i = pl.program_id(0)
    h = cf.shape[1]
    e = xf_ref.shape[2]

    @pl.when(i == 0)
    def _():
        col = lax.broadcasted_iota(jnp.int32, xhf.shape, 1)
        ones_col = jnp.where(col == h + e, 1.0, 0.0).astype(jnp.bfloat16)
        xhf[...] = ones_col
        xhb[...] = ones_col
        cf[...] = jnp.zeros_like(cf)
        cb[...] = jnp.zeros_like(cb)

    def step(x_blk, xh, wc, c_ref):
        xh[:, h:h + e] = x_blk.astype(jnp.bfloat16)
        g = jnp.dot(xh[...], wc[...], preferred_element_type=jnp.float32)
        ti = jnp.tanh(g[:, 0:h])
        tf_ = jnp.tanh(g[:, h:2 * h])
        tg = jnp.tanh(g[:, 2 * h:3 * h])
        to = jnp.tanh(g[:, 3 * h:4 * h])
        c_new = (0.5 * tf_ + 0.5) * c_ref[...] + (0.5 * ti + 0.5) * tg
        h_new = (0.5 * to + 0.5) * jnp.tanh(c_new)
        c_ref[...] = c_new
        hb16 = h_new.astype(jnp.bfloat16)
        xh[:, 0:h] = hb16
        return hb16

    for u in range(_UT):
        hfn = step(xf_ref[u], xhf, wcf, cf)
        outf[u] = (jnp.dot(hfn, wlfr[...], preferred_element_type=jnp.float32)
                   + blr[...]).astype(jnp.bfloat16)
        hbn = step(xb_ref[_UT - 1 - u], xhb, wcb, cb)
        outb[_UT - 1 - u] = jnp.dot(
            hbn, wlbr[...],
            preferred_element_type=jnp.float32).astype(jnp.bfloat16)


def _bilstm(x, wcat_f, wcat_b, wlf, wlb, blin2):
    t_len, b_sz, e = x.shape
    h = wcat_f.shape[1] // 4
    ncls = wlf.shape[1]

    nb = t_len // _UT
    full = lambda a: pl.BlockSpec(a.shape, lambda i: (0,) * a.ndim)
    outf, outb = pl.pallas_call(
        _bilstm_body,
        grid=(nb,),
        in_specs=[
            pl.BlockSpec((_UT, b_sz, e), lambda i: (i, 0, 0)),
            pl.BlockSpec((_UT, b_sz, e), lambda i: (nb - 1 - i, 0, 0)),
            full(wcat_f), full(wcat_b),
            full(wlf), full(wlb), full(blin2),
        ],
        out_specs=[
            pl.BlockSpec((_UT, b_sz, ncls), lambda i: (i, 0, 0)),
            pl.BlockSpec((_UT, b_sz, ncls), lambda i: (nb - 1 - i, 0, 0)),
        ],
        out_shape=[
            jax.ShapeDtypeStruct((t_len, b_sz, ncls), jnp.bfloat16),
            jax.ShapeDtypeStruct((t_len, b_sz, ncls), jnp.bfloat16),
        ],
        scratch_shapes=[
            pltpu.VMEM((b_sz, 2 * h), jnp.bfloat16),
            pltpu.VMEM((b_sz, 2 * h), jnp.bfloat16),
            pltpu.VMEM((b_sz, h), jnp.float32),
            pltpu.VMEM((b_sz, h), jnp.float32),
        ],
        compiler_params=pltpu.CompilerParams(
            dimension_semantics=("arbitrary",)),
    )(x, x, wcat_f, wcat_b, wlf, wlb, blin2)
    return outf, outb


def _combine_body(ncls, f_ref, b_ref, o_ref):
    s = f_ref[...].astype(jnp.float32) + b_ref[...].astype(jnp.float32)
    o_ref[...] = jnp.transpose(s, (1, 0, 2))[:, :, :ncls]


def _combine(outf, outb, ncls):
    t_len, b_sz, npad = outf.shape
    bb = 64
    return pl.pallas_call(
        functools.partial(_combine_body, ncls),
        grid=(b_sz // bb,),
        in_specs=[
            pl.BlockSpec((t_len, bb, npad), lambda i: (0, i, 0)),
            pl.BlockSpec((t_len, bb, npad), lambda i: (0, i, 0)),
        ],
        out_specs=pl.BlockSpec((bb, t_len, ncls), lambda i: (i, 0, 0)),
        out_shape=jax.ShapeDtypeStruct((b_sz, t_len, ncls), jnp.float32),
    )(outf, outb)


def kernel(inputs, emb, Wih_f, Whh_f, bih_f, bhh_f,
           Wih_b, Whh_b, bih_b, bhh_b, Wlin, blin):
    b_sz, t_len = inputs.shape
    e = emb.shape[1]
    h = Whh_f.shape[1]

    idx2d = inputs.T.reshape(-1).astype(jnp.int32)

    x = _sc_gather(emb.astype(jnp.bfloat16), idx2d).reshape(t_len, b_sz, e)

    bf16 = lambda a: a.astype(jnp.bfloat16)
    ncls = Wlin.shape[0]
    npad = 64
    pad = lambda a: jnp.pad(a, ((0, 0), (0, npad - ncls)))

    def wcat(Wih, Whh, bih, bhh):
        g4 = Whh.shape[0]
        zrows = 2 * h - (h + e + 1)
        m = jnp.concatenate([
            Whh.T, Wih.T, (bih + bhh).reshape(1, -1),
            jnp.zeros((zrows, g4), jnp.float32),
        ], axis=0)
        col = jnp.arange(g4)
        s = jnp.where((col >= 2 * h) & (col < 3 * h), 1.0, 0.5)
        return bf16(m * s[None, :])

    outf, outb = _bilstm(
        x,
        wcat(Wih_f, Whh_f, bih_f, bhh_f),
        wcat(Wih_b, Whh_b, bih_b, bhh_b),
        bf16(pad(Wlin[:, :h].T)), bf16(pad(Wlin[:, h:].T)),
        pad(blin.reshape(1, -1)),
    )
    return _combine(outf, outb, ncls)


# speedup vs baseline: 1.1326x; 1.1326x over previous
import functools

import jax
import jax.numpy as jnp
from jax import lax
from jax.experimental import pallas as pl
from jax.experimental.pallas import tpu as pltpu
from jax.experimental.pallas import tpu_sc as plsc


def _sc_gather(table, idx2d):
    rows_total = idx2d.shape[0] // 128
    d = table.shape[1]
    n_workers = 32
    rows_w = rows_total // n_workers
    streams = 5
    chunk_rows = streams * 128
    n_chunks = rows_w // streams
    n = rows_total * 128

    mesh = plsc.VectorSubcoreMesh(core_axis_name="c", subcore_axis_name="s")

    @functools.partial(
        pl.kernel,
        mesh=mesh,
        out_type=jax.ShapeDtypeStruct((n, d), table.dtype),
        scratch_types=[
            pltpu.VMEM((rows_w * 128,), jnp.int32),
            pltpu.VMEM((2 * chunk_rows, d), table.dtype),
            pltpu.SemaphoreType.DMA,
            pltpu.SemaphoreType.DMA,
        ],
        compiler_params=pltpu.CompilerParams(use_tc_tiling_on_sc=False),
    )
    def gather_kernel(idx_hbm, tab_hbm, out_hbm, idx_v, rows_v, sem_g, sem_s):
        wid = lax.axis_index("s") * 2 + lax.axis_index("c")
        out_base = wid * rows_w * 128
        pltpu.sync_copy(idx_hbm.at[pl.ds(out_base, rows_w * 128)], idx_v)

        def scatter_desc(i, cur):
            return pltpu.make_async_copy(
                rows_v.at[pl.ds(cur * chunk_rows, chunk_rows)],
                out_hbm.at[pl.ds(out_base + i * chunk_rows, chunk_rows)],
                sem_s)

        def body(i, carry):
            cur = lax.rem(i, 2)

            @pl.when(i >= 2)
            def _():
                scatter_desc(i - 2, cur).wait()

            handles = [
                pltpu.async_copy(
                    tab_hbm.at[idx_v.at[pl.ds((i * streams + j) * 128, 128)]],
                    rows_v.at[pl.ds(cur * chunk_rows + j * 128, 128)],
                    sem_g)
                for j in range(streams)
            ]
            for hnd in handles:
                hnd.wait()
            scatter_desc(i, cur).start()
            return carry

        lax.fori_loop(0, n_chunks, body, 0)
        scatter_desc(n_chunks - 2, (n_chunks - 2) % 2).wait()
        scatter_desc(n_chunks - 1, (n_chunks - 1) % 2).wait()

    return gather_kernel(idx2d, table)


def _sigmoid(x):
    return 0.5 * jnp.tanh(0.5 * x) + 0.5


_UT = 8


def _bilstm_body(xf_ref, xb_ref, wcf, wcb, wlfr, wlbr, blr,
                 outf, outb, xhf, xhb, cf, cb):
    i = pl.program_id(0)
    h = cf.shape[1]
    e = xf_ref.shape[2]

    @pl.when(i == 0)
    def _():
        col = lax.broadcasted_iota(jnp.int32, xhf.shape, 1)
        ones_col = jnp.where(col == h + e, 1.0, 0.0).astype(jnp.bfloat16)
        xhf[...] = ones_col
        xhb[...] = ones_col
        cf[...] = jnp.zeros_like(cf)
        cb[...] = jnp.zeros_like(cb)

    def step(x_blk, xh, wc, c_ref):
        xh[:, h:h + e] = x_blk.astype(jnp.bfloat16)
        g = jnp.dot(xh[...], wc[...], preferred_element_type=jnp.float32)
        ti = jnp.tanh(g[:, 0:h])
        tf_ = jnp.tanh(g[:, h:2 * h])
        tg = jnp.tanh(g[:, 2 * h:3 * h])
        to = jnp.tanh(g[:, 3 * h:4 * h])
        c_new = (0.5 * tf_ + 0.5) * c_ref[...] + (0.5 * ti + 0.5) * tg
        h_new = (0.5 * to + 0.5) * jnp.tanh(c_new)
        c_ref[...] = c_new
        hb16 = h_new.astype(jnp.bfloat16)
        xh[:, 0:h] = hb16
        return hb16

    for u in range(_UT):
        hfn = step(xf_ref[u], xhf, wcf, cf)
        outf[u] = (jnp.dot(hfn, wlfr[...], preferred_element_type=jnp.float32)
                   + blr[...]).astype(jnp.bfloat16)
        hbn = step(xb_ref[_UT - 1 - u], xhb, wcb, cb)
        outb[_UT - 1 - u] = jnp.dot(
            hbn, wlbr[...],
            preferred_element_type=jnp.float32).astype(jnp.bfloat16)


def _bilstm(x, wcat_f, wcat_b, wlf, wlb, blin2):
    t_len, b_sz, e = x.shape
    h = wcat_f.shape[1] // 4
    ncls = wlf.shape[1]

    nb = t_len // _UT
    full = lambda a: pl.BlockSpec(a.shape, lambda i: (0,) * a.ndim)
    outf, outb = pl.pallas_call(
        _bilstm_body,
        grid=(nb,),
        in_specs=[
            pl.BlockSpec((_UT, b_sz, e), lambda i: (i, 0, 0)),
            pl.BlockSpec((_UT, b_sz, e), lambda i: (nb - 1 - i, 0, 0)),
            full(wcat_f), full(wcat_b),
            full(wlf), full(wlb), full(blin2),
        ],
        out_specs=[
            pl.BlockSpec((_UT, b_sz, ncls), lambda i: (i, 0, 0)),
            pl.BlockSpec((_UT, b_sz, ncls), lambda i: (nb - 1 - i, 0, 0)),
        ],
        out_shape=[
            jax.ShapeDtypeStruct((t_len, b_sz, ncls), jnp.bfloat16),
            jax.ShapeDtypeStruct((t_len, b_sz, ncls), jnp.bfloat16),
        ],
        scratch_shapes=[
            pltpu.VMEM((b_sz, 2 * h), jnp.bfloat16),
            pltpu.VMEM((b_sz, 2 * h), jnp.bfloat16),
            pltpu.VMEM((b_sz, h), jnp.float32),
            pltpu.VMEM((b_sz, h), jnp.float32),
        ],
        compiler_params=pltpu.CompilerParams(
            dimension_semantics=("arbitrary",)),
    )(x, x, wcat_f, wcat_b, wlf, wlb, blin2)
    return outf, outb


def _combine_body(ncls, f_ref, b_ref, o_ref):
    s = f_ref[...].astype(jnp.float32) + b_ref[...].astype(jnp.float32)
    o_ref[...] = jnp.transpose(s, (1, 0, 2))[:, :, :ncls]


def _combine(outf, outb, ncls):
    t_len, b_sz, npad = outf.shape
    bb = 64
    return pl.pallas_call(
        functools.partial(_combine_body, ncls),
        grid=(b_sz // bb,),
        in_specs=[
            pl.BlockSpec((t_len, bb, npad), lambda i: (0, i, 0)),
            pl.BlockSpec((t_len, bb, npad), lambda i: (0, i, 0)),
        ],
        out_specs=pl.BlockSpec((bb, t_len, ncls), lambda i: (i, 0, 0)),
        out_shape=jax.ShapeDtypeStruct((b_sz, t_len, ncls), jnp.float32),
    )(outf, outb)


def kernel(inputs, emb, Wih_f, Whh_f, bih_f, bhh_f,
           Wih_b, Whh_b, bih_b, bhh_b, Wlin, blin):
    b_sz, t_len = inputs.shape
    e = emb.shape[1]
    h = Whh_f.shape[1]

    idx2d = inputs.T.reshape(-1).astype(jnp.int32)

    x = _sc_gather(emb, idx2d).reshape(t_len, b_sz, e)

    bf16 = lambda a: a.astype(jnp.bfloat16)
    ncls = Wlin.shape[0]
    npad = 64
    pad = lambda a: jnp.pad(a, ((0, 0), (0, npad - ncls)))

    def wcat(Wih, Whh, bih, bhh):
        g4 = Whh.shape[0]
        zrows = 2 * h - (h + e + 1)
        m = jnp.concatenate([
            Whh.T, Wih.T, (bih + bhh).reshape(1, -1),
            jnp.zeros((zrows, g4), jnp.float32),
        ], axis=0)
        col = jnp.arange(g4)
        s = jnp.where((col >= 2 * h) & (col < 3 * h), 1.0, 0.5)
        return bf16(m * s[None, :])

    outf, outb = _bilstm(
        x,
        wcat(Wih_f, Whh_f, bih_f, bhh_f),
        wcat(Wih_b, Whh_b, bih_b, bhh_b),
        bf16(pad(Wlin[:, :h].T)), bf16(pad(Wlin[:, h:].T)),
        pad(blin.reshape(1, -1)),
    )
    return _combine(outf, outb, ncls)
